# baseline (speedup 1.0000x reference)
import functools

import jax
import jax.numpy as jnp
from jax import lax
from jax.experimental import pallas as pl
from jax.experimental.pallas import tpu as pltpu
from jax.experimental.pallas import tpu_sc as plsc

D_VOCAB = 100000
D_MODEL = 768
NTOK = 4 * 2048

_NC = 2
_NS = 16
_NW = _NC * _NS
_L = 16
_D_PER_W = D_MODEL // _NW

_mesh = plsc.VectorSubcoreMesh(core_axis_name="c", subcore_axis_name="s")


@functools.partial(
    pl.kernel,
    mesh=_mesh,
    compiler_params=pltpu.CompilerParams(needs_layout_passes=False),
    out_type=jax.ShapeDtypeStruct((D_MODEL, NTOK), jnp.float32),
    scratch_types=[
        pltpu.VMEM((NTOK,), jnp.int32),
        pltpu.VMEM((D_VOCAB,), jnp.float32),
        pltpu.VMEM((NTOK,), jnp.float32),
    ],
)
def _sc_gather(x_hbm, w_hbm, out_hbm, tok_v, row_v, o_v):
    wid = lax.axis_index("s") * _NC + lax.axis_index("c")
    pltpu.sync_copy(x_hbm, tok_v)

    def d_loop(j, carry):
        d = wid * _D_PER_W + j
        pltpu.sync_copy(w_hbm.at[d], row_v)

        def step(i, c):
            idx = tok_v[pl.ds(i * _L, _L)]
            o_v[pl.ds(i * _L, _L)] = plsc.load_gather(row_v, [idx])
            return c

        lax.fori_loop(0, NTOK // _L, step, 0, unroll=4)
        pltpu.sync_copy(o_v, out_hbm.at[d])
        return carry

    lax.fori_loop(0, _D_PER_W, d_loop, 0)


def _transpose_body(in_ref, out_ref):
    out_ref[...] = in_ref[...].T


_TBLK = 512


def _tc_transpose(out_t):
    return pl.pallas_call(
        _transpose_body,
        grid=(NTOK // _TBLK,),
        in_specs=[pl.BlockSpec((D_MODEL, _TBLK), lambda i: (0, i))],
        out_specs=pl.BlockSpec((_TBLK, D_MODEL), lambda i: (i, 0)),
        out_shape=jax.ShapeDtypeStruct((NTOK, D_MODEL), jnp.float32),
    )(out_t)


def kernel(x, W_E):
    b, p = x.shape
    xf = x.reshape(-1).astype(jnp.int32)
    out_t = _sc_gather(xf, W_E)
    out = _tc_transpose(out_t)
    return out.reshape(b, p, D_MODEL)


# speedup vs baseline: 14.3795x; 14.3795x over previous
import functools

import jax
import jax.numpy as jnp
from jax import lax
from jax.experimental import pallas as pl
from jax.experimental.pallas import tpu as pltpu
from jax.experimental.pallas import tpu_sc as plsc

D_VOCAB = 100000
D_MODEL = 768
NTOK = 4 * 2048

_NC = 2
_NS = 16
_NW = _NC * _NS
_B_PER_W = NTOK // _NW
_CHUNK = 128

_mesh = plsc.VectorSubcoreMesh(core_axis_name="c", subcore_axis_name="s")


@functools.partial(
    pl.kernel,
    mesh=_mesh,
    compiler_params=pltpu.CompilerParams(needs_layout_passes=False),
    out_type=jax.ShapeDtypeStruct((NTOK, D_MODEL), jnp.float32),
    scratch_types=[
        pltpu.VMEM((_CHUNK,), jnp.int32),
        pltpu.VMEM((_CHUNK, D_MODEL), jnp.float32),
        pltpu.SemaphoreType.DMA,
    ],
)
def _sc_gather(x_hbm, wt_hbm, out_hbm, idx_v, rows_v, sem):
    wid = lax.axis_index("s") * _NC + lax.axis_index("c")
    base = wid * _B_PER_W

    def chunk(i, carry):
        off = base + i * _CHUNK
        pltpu.sync_copy(x_hbm.at[pl.ds(off, _CHUNK)], idx_v)
        pltpu.async_copy(wt_hbm.at[idx_v], rows_v, sem).wait()
        pltpu.sync_copy(rows_v, out_hbm.at[pl.ds(off, _CHUNK)])
        return carry

    lax.fori_loop(0, _B_PER_W // _CHUNK, chunk, 0)


def kernel(x, W_E):
    b, p = x.shape
    xf = x.reshape(-1)
    out = _sc_gather(xf, W_E.T)
    return out.reshape(b, p, D_MODEL)
